# baseline (speedup 1.0000x reference)
import functools

import jax
import jax.numpy as jnp
from jax import lax
from jax.experimental import pallas as pl
from jax.experimental.pallas import tpu as pltpu
from jax.experimental.pallas import tpu_sc as plsc

_L = 16
_UNROLL = 16


@functools.lru_cache(maxsize=None)
def _make_gather(seq, batch, d, vocab):
    mesh = plsc.VectorSubcoreMesh(core_axis_name="c", subcore_axis_name="s")
    nc = mesh.num_cores
    num_workers = nc * mesh.num_subcores
    d_per_w = d // num_workers
    steps = batch // (_L * _UNROLL)

    @functools.partial(
        pl.kernel,
        mesh=mesh,
        out_type=jax.ShapeDtypeStruct((seq, d, batch), jnp.float32),
        compiler_params=pltpu.CompilerParams(
            use_tc_tiling_on_sc=True, needs_layout_passes=False
        ),
        scratch_types=[
            pltpu.VMEM((vocab,), jnp.float32),
            pltpu.VMEM((batch,), jnp.int32),
            pltpu.VMEM((batch,), jnp.int32),
            pltpu.VMEM((batch,), jnp.float32),
            pltpu.VMEM((batch,), jnp.float32),
            pltpu.SemaphoreType.DMA,
            pltpu.SemaphoreType.DMA,
            pltpu.SemaphoreType.DMA,
            pltpu.SemaphoreType.DMA,
        ],
    )
    def gather(ids_hbm, table_hbm, out_hbm, row_v, ib0, ib1, ob0, ob1,
               si0, si1, so0, so1):
        wid = lax.axis_index("s") * nc + lax.axis_index("c")
        ibs = (ib0, ib1)
        obs = (ob0, ob1)
        sis = (si0, si1)
        sos = (so0, so1)

        def compute(ib, ob):
            @plsc.parallel_loop(0, batch, step=_L, unroll=_UNROLL)
            def _(off):
                idx = ib[pl.ds(off, _L)]
                ob[pl.ds(off, _L)] = plsc.load_gather(row_v, [idx])

        def run_feature(dd):
            pltpu.sync_copy(table_hbm.at[dd], row_v)
            pltpu.async_copy(ids_hbm.at[0], ibs[0], sis[0])

            def s_iter(i, carry):
                for b in range(2):
                    s = i * 2 + b
                    nb = 1 - b
                    pltpu.make_async_copy(ids_hbm.at[s], ibs[b], sis[b]).wait()
                    if b == 0:
                        pltpu.async_copy(ids_hbm.at[s + 1], ibs[nb], sis[nb])
                    else:
                        @pl.when(s + 1 < seq)
                        def _():
                            pltpu.async_copy(ids_hbm.at[s + 1], ibs[nb], sis[nb])

                    @pl.when(i >= 1)
                    def _():
                        pltpu.make_async_copy(
                            obs[b], out_hbm.at[s - 2, dd], sos[b]
                        ).wait()

                    compute(ibs[b], obs[b])
                    pltpu.async_copy(obs[b], out_hbm.at[s, dd], sos[b])
                return carry

            lax.fori_loop(0, seq // 2, s_iter, 0)
            for b in range(2):
                pltpu.make_async_copy(obs[b], out_hbm.at[seq - 2 + b, dd],
                                      sos[b]).wait()

        for dp in range(d_per_w):
            run_feature(wid + num_workers * dp)

    return gather


def kernel(ids, table):
    batch, seq = ids.shape
    vocab, d = table.shape
    ids_t = jnp.transpose(ids.astype(jnp.int32))
    table_t = jnp.transpose(table)

    info = plsc.get_sparse_core_info()
    num_workers = info.num_cores * info.num_subcores
    assert d % num_workers == 0 and seq % 2 == 0
    assert batch % (_L * _UNROLL) == 0
    out_t = _make_gather(seq, batch, d, vocab)(ids_t, table_t)
    return jnp.transpose(out_t, (2, 0, 1))


# speedup vs baseline: 1.2886x; 1.2886x over previous
import functools

import jax
import jax.numpy as jnp
from jax import lax
from jax.experimental import pallas as pl
from jax.experimental.pallas import tpu as pltpu
from jax.experimental.pallas import tpu_sc as plsc

_L = 16
_UNROLL = 8


@functools.lru_cache(maxsize=None)
def _make_gather(seq, batch, d, vocab):
    mesh = plsc.VectorSubcoreMesh(core_axis_name="c", subcore_axis_name="s")
    nc = mesh.num_cores
    num_workers = nc * mesh.num_subcores
    d_per_w = d // num_workers
    assert seq % 6 == 2

    @functools.partial(
        pl.kernel,
        mesh=mesh,
        out_type=jax.ShapeDtypeStruct((seq, d, batch), jnp.float32),
        compiler_params=pltpu.CompilerParams(
            use_tc_tiling_on_sc=True, needs_layout_passes=False
        ),
        scratch_types=[
            pltpu.VMEM((vocab,), jnp.float32),
            pltpu.VMEM((batch,), jnp.int32),
            pltpu.VMEM((batch,), jnp.int32),
            pltpu.VMEM((batch,), jnp.int32),
            pltpu.VMEM((batch,), jnp.float32),
            pltpu.VMEM((batch,), jnp.float32),
            pltpu.SemaphoreType.DMA,
            pltpu.SemaphoreType.DMA,
            pltpu.SemaphoreType.DMA,
            pltpu.SemaphoreType.DMA,
            pltpu.SemaphoreType.DMA,
        ],
    )
    def gather(ids_hbm, table_hbm, out_hbm, row_v, ib0, ib1, ib2, ob0, ob1,
               si0, si1, si2, so0, so1):
        wid = lax.axis_index("s") * nc + lax.axis_index("c")
        ibs = (ib0, ib1, ib2)
        obs = (ob0, ob1)
        sis = (si0, si1, si2)
        sos = (so0, so1)

        def compute(ib, ob):
            @plsc.parallel_loop(0, batch, step=_L, unroll=_UNROLL)
            def _(off):
                idx = ib[pl.ds(off, _L)]
                ob[pl.ds(off, _L)] = plsc.load_gather(row_v, [idx])

        def run_feature(dd):
            pltpu.sync_copy(table_hbm.at[dd], row_v)
            for p in range(2):
                pltpu.async_copy(ids_hbm.at[p], ibs[p], sis[p])

            def body(s, bi, bo, prefetch, reclaim_always, i):
                pltpu.make_async_copy(ids_hbm.at[s], ibs[bi], sis[bi]).wait()
                if prefetch:
                    ni = (bi + 2) % 3
                    pltpu.async_copy(ids_hbm.at[s + 2], ibs[ni], sis[ni])

                def reclaim():
                    pltpu.make_async_copy(
                        obs[bo], out_hbm.at[s - 2, dd], sos[bo]
                    ).wait()

                if reclaim_always:
                    reclaim()
                else:
                    @pl.when(i >= 1)
                    def _():
                        reclaim()

                compute(ibs[bi], obs[bo])
                pltpu.async_copy(obs[bo], out_hbm.at[s, dd], sos[bo])

            def s_iter(i, carry):
                for u in range(6):
                    body(i * 6 + u, u % 3, u % 2, True, u >= 2, i)
                return carry

            lax.fori_loop(0, (seq - 2) // 6, s_iter, 0)
            for s in (seq - 2, seq - 1):
                body(s, s % 3, s % 2, False, True, 0)
            for b in range(2):
                pltpu.make_async_copy(obs[b], out_hbm.at[seq - 2 + b, dd],
                                      sos[b]).wait()

        for dp in range(d_per_w):
            run_feature(wid + num_workers * dp)

    return gather


def kernel(ids, table):
    batch, seq = ids.shape
    vocab, d = table.shape
    ids_t = jnp.transpose(ids.astype(jnp.int32))
    table_t = jnp.transpose(table)

    info = plsc.get_sparse_core_info()
    num_workers = info.num_cores * info.num_subcores
    assert d % num_workers == 0 and seq % 2 == 0
    assert batch % (_L * _UNROLL) == 0
    out_t = _make_gather(seq, batch, d, vocab)(ids_t, table_t)
    return jnp.transpose(out_t, (2, 0, 1))


# speedup vs baseline: 1.3537x; 1.0505x over previous
import functools

import jax
import jax.numpy as jnp
from jax import lax
from jax.experimental import pallas as pl
from jax.experimental.pallas import tpu as pltpu
from jax.experimental.pallas import tpu_sc as plsc

_L = 16
_UNROLL = 8


@functools.lru_cache(maxsize=None)
def _make_gather(seq, batch, d, vocab):
    mesh = plsc.VectorSubcoreMesh(core_axis_name="c", subcore_axis_name="s")
    nc = mesh.num_cores
    num_workers = nc * mesh.num_subcores
    d_per_w = d // num_workers
    ni_, no_, u_ = 4, 3, 12
    assert seq % u_ == 2

    @functools.partial(
        pl.kernel,
        mesh=mesh,
        out_type=jax.ShapeDtypeStruct((seq, d, batch), jnp.float32),
        compiler_params=pltpu.CompilerParams(
            use_tc_tiling_on_sc=True, needs_layout_passes=False
        ),
        scratch_types=[
            pltpu.VMEM((vocab,), jnp.float32),
        ]
        + [pltpu.VMEM((batch,), jnp.int32) for _ in range(ni_)]
        + [pltpu.VMEM((batch,), jnp.float32) for _ in range(no_)]
        + [pltpu.SemaphoreType.DMA for _ in range(ni_ + no_)],
    )
    def gather(ids_hbm, table_hbm, out_hbm, row_v, *bufs):
        wid = lax.axis_index("s") * nc + lax.axis_index("c")
        ibs = bufs[:ni_]
        obs = bufs[ni_:ni_ + no_]
        sis = bufs[ni_ + no_:2 * ni_ + no_]
        sos = bufs[2 * ni_ + no_:]

        def compute(ib, ob):
            @plsc.parallel_loop(0, batch, step=_L, unroll=_UNROLL)
            def _(off):
                idx = ib[pl.ds(off, _L)]
                ob[pl.ds(off, _L)] = plsc.load_gather(row_v, [idx])

        def run_feature(dd):
            pltpu.sync_copy(table_hbm.at[dd], row_v)
            for p in range(ni_ - 1):
                pltpu.async_copy(ids_hbm.at[p], ibs[p], sis[p])

            def body(s, bi, bo, prefetch, reclaim_always, i):
                pltpu.make_async_copy(ids_hbm.at[s], ibs[bi], sis[bi]).wait()
                if prefetch is not None:
                    nxt = (bi + ni_ - 1) % ni_

                    def fire():
                        pltpu.async_copy(
                            ids_hbm.at[s + ni_ - 1], ibs[nxt], sis[nxt]
                        )

                    if prefetch is True:
                        fire()
                    else:
                        pl.when(prefetch)(fire)

                def reclaim():
                    pltpu.make_async_copy(
                        obs[bo], out_hbm.at[s - no_, dd], sos[bo]
                    ).wait()

                if reclaim_always:
                    reclaim()
                else:
                    @pl.when(i >= 1)
                    def _():
                        reclaim()

                compute(ibs[bi], obs[bo])
                pltpu.async_copy(obs[bo], out_hbm.at[s, dd], sos[bo])

            n_iter = (seq - 2) // u_

            def s_iter(i, carry):
                for u in range(u_):
                    s = i * u_ + u
                    pf = True
                    if (n_iter - 1) * u_ + u + ni_ - 1 >= seq:
                        pf = i < n_iter - 1
                    body(s, u % ni_, u % no_, pf, u >= no_, i)
                return carry

            lax.fori_loop(0, n_iter, s_iter, 0)
            for s in (seq - 2, seq - 1):
                body(s, s % ni_, s % no_, None, True, 0)
            for s in range(seq - no_, seq):
                pltpu.make_async_copy(obs[s % no_], out_hbm.at[s, dd],
                                      sos[s % no_]).wait()

        for dp in range(d_per_w):
            run_feature(wid + num_workers * dp)

    return gather


def kernel(ids, table):
    batch, seq = ids.shape
    vocab, d = table.shape
    ids_t = jnp.transpose(ids.astype(jnp.int32))
    table_t = jnp.transpose(table)

    info = plsc.get_sparse_core_info()
    num_workers = info.num_cores * info.num_subcores
    assert d % num_workers == 0 and seq % 2 == 0
    assert batch % (_L * _UNROLL) == 0
    out_t = _make_gather(seq, batch, d, vocab)(ids_t, table_t)
    return jnp.transpose(out_t, (2, 0, 1))


# speedup vs baseline: 1.9567x; 1.4455x over previous
import functools

import jax
import jax.numpy as jnp
from jax import lax
from jax.experimental import pallas as pl
from jax.experimental.pallas import tpu as pltpu
from jax.experimental.pallas import tpu_sc as plsc

_L = 16
_UNROLL = 8


@functools.lru_cache(maxsize=None)
def _make_gather(seq, batch, d, vocab):
    mesh = plsc.VectorSubcoreMesh(core_axis_name="c", subcore_axis_name="s")
    nc = mesh.num_cores
    num_workers = nc * mesh.num_subcores
    d_per_w = d // num_workers
    ni_, no_, u_ = 2, 2, 4
    assert (seq - 2) % u_ == 0

    @functools.partial(
        pl.kernel,
        mesh=mesh,
        out_type=jax.ShapeDtypeStruct((seq, d, batch), jnp.float32),
        compiler_params=pltpu.CompilerParams(
            use_tc_tiling_on_sc=True, needs_layout_passes=False
        ),
        scratch_types=[
            pltpu.VMEM((vocab,), jnp.float32),
            pltpu.VMEM_SHARED((seq * batch,), jnp.int32),
        ]
        + [pltpu.VMEM((batch,), jnp.int32) for _ in range(ni_)]
        + [pltpu.VMEM((batch,), jnp.float32) for _ in range(no_)]
        + [pltpu.SemaphoreType.DMA for _ in range(ni_ + no_ + 1)],
    )
    def gather(ids_hbm, table_hbm, out_hbm, row_v, ids_sh, *bufs):
        wid = lax.axis_index("s") * nc + lax.axis_index("c")
        ibs = bufs[:ni_]
        obs = bufs[ni_:ni_ + no_]
        sis = bufs[ni_ + no_:2 * ni_ + no_]
        sos = bufs[2 * ni_ + no_:2 * (ni_ + no_)]
        st_sem = bufs[2 * (ni_ + no_)]

        sid = lax.axis_index("s")
        n_share = (seq + mesh.num_subcores - 1) // mesh.num_subcores
        for k in range(n_share):
            r = sid + mesh.num_subcores * k

            @pl.when(r < seq)
            def _():
                pltpu.async_copy(
                    ids_hbm.at[r], ids_sh.at[pl.ds(r * batch, batch)], st_sem
                )

        for k in range(n_share):
            r = sid + mesh.num_subcores * k

            @pl.when(r < seq)
            def _():
                pltpu.make_async_copy(
                    ids_hbm.at[r], ids_sh.at[pl.ds(r * batch, batch)], st_sem
                ).wait()

        plsc.subcore_barrier()

        def compute(ib, ob):
            @plsc.parallel_loop(0, batch, step=_L, unroll=_UNROLL)
            def _(off):
                idx = ib[pl.ds(off, _L)]
                ob[pl.ds(off, _L)] = plsc.load_gather(row_v, [idx])

        def run_feature(dd):
            pltpu.sync_copy(table_hbm.at[dd], row_v)

            def ids_row(s):
                return ids_sh.at[pl.ds(s * batch, batch)]

            for p in range(ni_ - 1):
                pltpu.async_copy(ids_row(p), ibs[p], sis[p])

            def body(s, bi, bo, prefetch, reclaim_always, i):
                pltpu.make_async_copy(ids_row(s), ibs[bi], sis[bi]).wait()
                if prefetch is not None:
                    nxt = (bi + ni_ - 1) % ni_

                    def fire():
                        pltpu.async_copy(
                            ids_row(s + ni_ - 1), ibs[nxt], sis[nxt]
                        )

                    if prefetch is True:
                        fire()
                    else:
                        pl.when(prefetch)(fire)

                def reclaim():
                    pltpu.make_async_copy(
                        obs[bo], out_hbm.at[s - no_, dd], sos[bo]
                    ).wait()

                if reclaim_always:
                    reclaim()
                else:
                    @pl.when(i >= 1)
                    def _():
                        reclaim()

                compute(ibs[bi], obs[bo])
                pltpu.async_copy(obs[bo], out_hbm.at[s, dd], sos[bo])

            n_iter = (seq - 2) // u_

            def s_iter(i, carry):
                for u in range(u_):
                    s = i * u_ + u
                    pf = True
                    if (n_iter - 1) * u_ + u + ni_ - 1 >= seq:
                        pf = i < n_iter - 1
                    body(s, u % ni_, u % no_, pf, u >= no_, i)
                return carry

            lax.fori_loop(0, n_iter, s_iter, 0)
            for s in (seq - 2, seq - 1):
                pf = True if s + ni_ - 1 < seq else None
                body(s, s % ni_, s % no_, pf, True, 0)
            for s in range(seq - no_, seq):
                pltpu.make_async_copy(obs[s % no_], out_hbm.at[s, dd],
                                      sos[s % no_]).wait()

        for dp in range(d_per_w):
            run_feature(wid + num_workers * dp)

    return gather


def kernel(ids, table):
    batch, seq = ids.shape
    vocab, d = table.shape
    ids_t = jnp.transpose(ids.astype(jnp.int32))
    table_t = jnp.transpose(table)

    info = plsc.get_sparse_core_info()
    num_workers = info.num_cores * info.num_subcores
    assert d % num_workers == 0 and seq % 2 == 0
    assert batch % (_L * _UNROLL) == 0
    out_t = _make_gather(seq, batch, d, vocab)(ids_t, table_t)
    return jnp.transpose(out_t, (2, 0, 1))


# speedup vs baseline: 2.0161x; 1.0303x over previous
import functools

import jax
import jax.numpy as jnp
from jax import lax
from jax.experimental import pallas as pl
from jax.experimental.pallas import tpu as pltpu
from jax.experimental.pallas import tpu_sc as plsc

_L = 16
_UNROLL = 8


@functools.lru_cache(maxsize=None)
def _make_gather(seq, batch, d, vocab):
    mesh = plsc.VectorSubcoreMesh(core_axis_name="c", subcore_axis_name="s")
    nc = mesh.num_cores
    num_workers = nc * mesh.num_subcores
    d_per_w = d // num_workers
    ni_, no_, u_ = 2, 2, 4
    assert (seq - 2) % u_ == 0

    @functools.partial(
        pl.kernel,
        mesh=mesh,
        out_type=jax.ShapeDtypeStruct((seq, d, batch), jnp.float32),
        compiler_params=pltpu.CompilerParams(
            use_tc_tiling_on_sc=True, needs_layout_passes=False
        ),
        scratch_types=[
            pltpu.VMEM((vocab,), jnp.float32),
            pltpu.VMEM_SHARED((seq * batch,), jnp.int32),
        ]
        + [pltpu.VMEM((batch,), jnp.int32) for _ in range(ni_)]
        + [pltpu.VMEM((batch,), jnp.float32) for _ in range(no_)]
        + [pltpu.SemaphoreType.DMA for _ in range(ni_ + no_ + 2)],
    )
    def gather(ids_hbm, table_hbm, out_hbm, row_v, ids_sh, *bufs):
        wid = lax.axis_index("s") * nc + lax.axis_index("c")
        ibs = bufs[:ni_]
        obs = bufs[ni_:ni_ + no_]
        sis = bufs[ni_ + no_:2 * ni_ + no_]
        sos = bufs[2 * ni_ + no_:2 * (ni_ + no_)]
        st_sem = bufs[2 * (ni_ + no_)]
        row_sem = bufs[2 * (ni_ + no_) + 1]

        pltpu.async_copy(table_hbm.at[wid], row_v, row_sem)

        sid = lax.axis_index("s")
        n_share = (seq + mesh.num_subcores - 1) // mesh.num_subcores
        for k in range(n_share):
            r = sid + mesh.num_subcores * k

            @pl.when(r < seq)
            def _():
                pltpu.async_copy(
                    ids_hbm.at[r], ids_sh.at[pl.ds(r * batch, batch)], st_sem
                )

        for k in range(n_share):
            r = sid + mesh.num_subcores * k

            @pl.when(r < seq)
            def _():
                pltpu.make_async_copy(
                    ids_hbm.at[r], ids_sh.at[pl.ds(r * batch, batch)], st_sem
                ).wait()

        plsc.subcore_barrier()

        def compute(ib, ob):
            @plsc.parallel_loop(0, batch, step=_L, unroll=_UNROLL)
            def _(off):
                idx = ib[pl.ds(off, _L)]
                ob[pl.ds(off, _L)] = plsc.load_gather(row_v, [idx])

        def run_feature(dd, fire_next):
            def ids_row(s):
                return ids_sh.at[pl.ds(s * batch, batch)]

            for p in range(ni_ - 1):
                pltpu.async_copy(ids_row(p), ibs[p], sis[p])
            pltpu.make_async_copy(table_hbm.at[dd], row_v, row_sem).wait()

            def body(s, bi, bo, prefetch, reclaim_always, i):
                pltpu.make_async_copy(ids_row(s), ibs[bi], sis[bi]).wait()
                if prefetch is not None:
                    nxt = (bi + ni_ - 1) % ni_

                    def fire():
                        pltpu.async_copy(
                            ids_row(s + ni_ - 1), ibs[nxt], sis[nxt]
                        )

                    if prefetch is True:
                        fire()
                    else:
                        pl.when(prefetch)(fire)

                def reclaim():
                    pltpu.make_async_copy(
                        obs[bo], out_hbm.at[s - no_, dd], sos[bo]
                    ).wait()

                if reclaim_always:
                    reclaim()
                else:
                    @pl.when(i >= 1)
                    def _():
                        reclaim()

                compute(ibs[bi], obs[bo])
                pltpu.async_copy(obs[bo], out_hbm.at[s, dd], sos[bo])

            n_iter = (seq - 2) // u_

            def s_iter(i, carry):
                for u in range(u_):
                    s = i * u_ + u
                    pf = True
                    if (n_iter - 1) * u_ + u + ni_ - 1 >= seq:
                        pf = i < n_iter - 1
                    body(s, u % ni_, u % no_, pf, u >= no_, i)
                return carry

            lax.fori_loop(0, n_iter, s_iter, 0)
            for s in (seq - 2, seq - 1):
                pf = True if s + ni_ - 1 < seq else None
                body(s, s % ni_, s % no_, pf, True, 0)
            if fire_next is not None:
                pltpu.async_copy(table_hbm.at[fire_next], row_v, row_sem)
            for s in range(seq - no_, seq):
                pltpu.make_async_copy(obs[s % no_], out_hbm.at[s, dd],
                                      sos[s % no_]).wait()

        for dp in range(d_per_w):
            dd = wid + num_workers * dp
            nxt = wid + num_workers * (dp + 1) if dp + 1 < d_per_w else None
            run_feature(dd, nxt)

    return gather


def kernel(ids, table):
    batch, seq = ids.shape
    vocab, d = table.shape
    ids_t = jnp.transpose(ids.astype(jnp.int32))
    table_t = jnp.transpose(table)

    info = plsc.get_sparse_core_info()
    num_workers = info.num_cores * info.num_subcores
    assert d % num_workers == 0 and seq % 2 == 0
    assert batch % (_L * _UNROLL) == 0
    out_t = _make_gather(seq, batch, d, vocab)(ids_t, table_t)
    return jnp.transpose(out_t, (2, 0, 1))
